# baseline (speedup 1.0000x reference)
import dataclasses
import functools

import jax
import jax.numpy as jnp
from jax import lax
from jax.experimental import pallas as pl
from jax.experimental.pallas import tpu as pltpu
from jax.experimental.pallas import tpu_sc as plsc

N_USERS = 1000000
N_ITEMS = 100000
N_FACTORS = 32
BATCH = 16384

NUM_CORES = 2
NUM_SUBCORES = 16
NUM_WORKERS = NUM_CORES * NUM_SUBCORES
B_PER_W = BATCH // NUM_WORKERS
IDX_CHUNK = 128
CHUNKS_PER_W = B_PER_W // IDX_CHUNK
LANES = 16


def _body(users_hbm, items_hbm, u_hbm, v_hbm, out_hbm,
          uidx, vidx, u_rows, v_rows, out_v, sem):
  wid = lax.axis_index("s") * NUM_CORES + lax.axis_index("c")

  pltpu.sync_copy(users_hbm.at[pl.ds(wid * CHUNKS_PER_W, CHUNKS_PER_W)], uidx)
  pltpu.sync_copy(items_hbm.at[pl.ds(wid * CHUNKS_PER_W, CHUNKS_PER_W)], vidx)

  copies = []
  for i in range(CHUNKS_PER_W):
    copies.append(pltpu.async_copy(
        u_hbm.at[uidx.at[i]], u_rows.at[pl.ds(i * IDX_CHUNK, IDX_CHUNK)], sem))
    copies.append(pltpu.async_copy(
        v_hbm.at[vidx.at[i]], v_rows.at[pl.ds(i * IDX_CHUNK, IDX_CHUNK)], sem))
  for c in copies:
    c.wait()

  @pl.loop(0, B_PER_W, step=LANES)
  def _(j):
    rows = lax.iota(jnp.int32, LANES) + j
    acc = jnp.zeros((LANES,), jnp.float32)
    for k in range(N_FACTORS):
      cols = jnp.full((LANES,), k, jnp.int32)
      uc = plsc.load_gather(u_rows, [rows, cols])
      vc = plsc.load_gather(v_rows, [rows, cols])
      acc = acc + uc * vc
    out_v[pl.ds(j, LANES)] = acc

  pltpu.sync_copy(out_v, out_hbm.at[pl.ds(wid * B_PER_W, B_PER_W)])


@jax.jit
def _pmf_sc(users, items, u_table, v_table):
  mesh = plsc.VectorSubcoreMesh(
      core_axis_name="c", subcore_axis_name="s",
      num_cores=NUM_CORES, num_subcores=NUM_SUBCORES)
  cp = pltpu.CompilerParams(use_tc_tiling_on_sc=False)
  if "needs_layout_passes" in pltpu.CompilerParams.__dataclass_fields__:
    cp = dataclasses.replace(cp, needs_layout_passes=False)
  run = pl.kernel(
      _body,
      out_type=jax.ShapeDtypeStruct((BATCH,), jnp.float32),
      mesh=mesh,
      scratch_types=[
          pltpu.VMEM((CHUNKS_PER_W, IDX_CHUNK), jnp.int32),
          pltpu.VMEM((CHUNKS_PER_W, IDX_CHUNK), jnp.int32),
          pltpu.VMEM((B_PER_W, N_FACTORS), jnp.float32),
          pltpu.VMEM((B_PER_W, N_FACTORS), jnp.float32),
          pltpu.VMEM((B_PER_W,), jnp.float32),
          pltpu.SemaphoreType.DMA,
      ],
      compiler_params=cp,
  )
  return run(users, items, u_table, v_table)


def kernel(users_index, items_index, U, V):
  users = users_index.astype(jnp.int32).reshape(BATCH // IDX_CHUNK, IDX_CHUNK)
  items = items_index.astype(jnp.int32).reshape(BATCH // IDX_CHUNK, IDX_CHUNK)
  return _pmf_sc(users, items, U, V)
